# baseline (speedup 1.0000x reference)
import functools

import jax
import jax.numpy as jnp
from jax import lax
from jax.experimental import pallas as pl
from jax.experimental.pallas import tpu as pltpu
from jax.experimental.pallas import tpu_sc as plsc

_EPS = 1e-15
_B = 16384
_D = 64
_WIN = 20
_NC = 2
_NS = 16
_NW = _NC * _NS
_BPW = _B // _NW
_BLK = 128
_HALF = 256
_IPOS = _WIN
_INEG = _WIN + 1


def _sc_body(pos_hbm, neg_hbm, ctxT_hbm, tab_hbm, opos_hbm, oneg_hbm,
             idx_all, pos_rows, neg_rows, acc, stage_p, stage_n,
             sem_idx, sem_g, sem_a, sem_o):
    wid = lax.axis_index("s") * _NC + lax.axis_index("c")
    base = wid * _BPW

    idx_cps = [
        pltpu.async_copy(pos_hbm.at[pl.ds(base, _BPW)], idx_all.at[_IPOS],
                         sem_idx),
        pltpu.async_copy(neg_hbm.at[pl.ds(base, _BPW)], idx_all.at[_INEG],
                         sem_idx),
    ]
    for w in range(_WIN):
        idx_cps.append(pltpu.async_copy(
            ctxT_hbm.at[w, pl.ds(base, _BPW)], idx_all.at[w], sem_idx))
    for c in idx_cps:
        c.wait()

    for half in range(_BPW // _HALF):
        hoff = half * _HALF
        g1 = []
        for j in range(_HALF // _BLK):
            src = pl.ds(hoff + j * _BLK, _BLK)
            dst = pl.ds(j * _BLK, _BLK)
            g1.append(pltpu.async_copy(
                tab_hbm.at[idx_all.at[_IPOS, src]], pos_rows.at[dst], sem_g))
            g1.append(pltpu.async_copy(
                tab_hbm.at[idx_all.at[_INEG, src]], neg_rows.at[dst], sem_g))
            g1.append(pltpu.async_copy(
                tab_hbm.at[idx_all.at[0, src]], acc.at[dst], sem_a))
        for c in g1:
            c.wait()
        g2 = []
        for j in range(_HALF // _BLK):
            src = pl.ds(hoff + j * _BLK, _BLK)
            dst = pl.ds(j * _BLK, _BLK)
            for w in range(1, _WIN):
                g2.append(pltpu.async_copy(
                    tab_hbm.at[idx_all.at[w, src]], acc.at[dst], sem_a,
                    add=True))
        for c in g2:
            c.wait()

        def elem(e, carry):
            pv = None
            nv = None
            for k in range(_D // 16):
                a = acc[e, pl.ds(_D + k * 16, 16)]
                p = pos_rows[e, pl.ds(k * 16, 16)] * a
                n = neg_rows[e, pl.ds(k * 16, 16)] * a
                pv = p if pv is None else pv + p
                nv = n if nv is None else nv + n
            row = (hoff + e) // 8
            lane = ((hoff + e) % 8) * 16
            stage_p[row, pl.ds(lane, 16)] = pv
            stage_n[row, pl.ds(lane, 16)] = nv
            return carry

        lax.fori_loop(0, _HALF, elem, 0)

    orow = wid * (_BPW // 8)
    o1 = pltpu.async_copy(stage_p, opos_hbm.at[pl.ds(orow, _BPW // 8)], sem_o)
    o2 = pltpu.async_copy(stage_n, oneg_hbm.at[pl.ds(orow, _BPW // 8)], sem_o)
    o1.wait()
    o2.wait()


_sc_cbow = functools.partial(
    pl.kernel,
    out_type=(jax.ShapeDtypeStruct((_B // 8, 128), jnp.float32),
              jax.ShapeDtypeStruct((_B // 8, 128), jnp.float32)),
    mesh=plsc.VectorSubcoreMesh(core_axis_name="c", subcore_axis_name="s",
                                num_cores=_NC, num_subcores=_NS),
    scratch_types=[
        pltpu.VMEM((_WIN + 2, _BPW), jnp.int32),
        pltpu.VMEM((_HALF, 2 * _D), jnp.float32),
        pltpu.VMEM((_HALF, 2 * _D), jnp.float32),
        pltpu.VMEM((_HALF, 2 * _D), jnp.float32),
        pltpu.VMEM((_BPW // 8, 128), jnp.float32),
        pltpu.VMEM((_BPW // 8, 128), jnp.float32),
        pltpu.SemaphoreType.DMA,
        pltpu.SemaphoreType.DMA,
        pltpu.SemaphoreType.DMA,
        pltpu.SemaphoreType.DMA,
    ],
    compiler_params=pltpu.CompilerParams(use_tc_tiling_on_sc=True),
)(_sc_body)


_N = 1000001
_CH = 512
_NCHUNK = -(-_N // _CH)


def _tc_tab(twT_ref, cwT_ref, tab_ref):
    tab_ref[...] = jnp.concatenate(
        [twT_ref[...].T, cwT_ref[...].T], axis=1)


_build_tab = pl.pallas_call(
    _tc_tab,
    grid=(_NCHUNK,),
    in_specs=[pl.BlockSpec((_D, _CH), lambda i: (0, i)),
              pl.BlockSpec((_D, _CH), lambda i: (0, i))],
    out_specs=pl.BlockSpec((_CH, 2 * _D), lambda i: (i, 0)),
    out_shape=jax.ShapeDtypeStruct((_N, 2 * _D), jnp.float32),
)


def _tc_finish(pp_ref, np_ref, out_ref):
    l = lax.broadcasted_iota(jnp.int32, (128, 8), 0)
    s = lax.broadcasted_iota(jnp.int32, (128, 8), 1)
    m = (l // 16 == s).astype(jnp.float32)
    ps = jnp.dot(pp_ref[...], m) * (1.0 / _WIN) + _EPS
    ns = jnp.dot(np_ref[...], m) * (1.0 / _WIN) + _EPS
    pos_score = -jax.nn.log_sigmoid(ps)
    neg_score = -jax.nn.log_sigmoid(1.0 - ns)
    out_ref[0, 0] = (jnp.sum(pos_score) + jnp.sum(neg_score)) * (1.0 / _B)


def kernel(pos_nodes, neg_nodes, context_nodes, target_weight, context_weight):
    pos = pos_nodes.astype(jnp.int32)
    neg = neg_nodes.astype(jnp.int32)
    ctxT = context_nodes.astype(jnp.int32).T
    tab = _build_tab(target_weight.T, context_weight.T)

    pp, nn = _sc_cbow(pos, neg, ctxT, tab)

    loss = pl.pallas_call(
        _tc_finish,
        out_shape=jax.ShapeDtypeStruct((1, 1), jnp.float32),
        out_specs=pl.BlockSpec(memory_space=pltpu.SMEM),
    )(pp, nn)
    return loss[0, 0]


# speedup vs baseline: 2.4426x; 2.4426x over previous
import functools

import jax
import jax.numpy as jnp
from jax import lax
from jax.experimental import pallas as pl
from jax.experimental.pallas import tpu as pltpu
from jax.experimental.pallas import tpu_sc as plsc

_EPS = 1e-15
_B = 16384
_D = 64
_WIN = 20
_NC = 2
_NS = 16
_NW = _NC * _NS
_BPW = _B // _NW
_BLK = 128
_HALF = 256
_IPOS = _WIN
_INEG = _WIN + 1


def _sc_body(pos_hbm, neg_hbm, ctxT_hbm, tab_hbm, opos_hbm, oneg_hbm,
             idx_all, pos_rows, neg_rows, acc, stage_p, stage_n,
             sem_idx, sem_g, sem_a, sem_o):
    wid = lax.axis_index("s") * _NC + lax.axis_index("c")
    base = wid * _BPW

    idx_cps = [
        pltpu.async_copy(pos_hbm.at[pl.ds(base, _BPW)], idx_all.at[_IPOS],
                         sem_idx),
        pltpu.async_copy(neg_hbm.at[pl.ds(base, _BPW)], idx_all.at[_INEG],
                         sem_idx),
    ]
    for w in range(_WIN):
        idx_cps.append(pltpu.async_copy(
            ctxT_hbm.at[w, pl.ds(base, _BPW)], idx_all.at[w], sem_idx))
    for c in idx_cps:
        c.wait()

    for half in range(_BPW // _HALF):
        hoff = half * _HALF
        g1 = []
        for j in range(_HALF // _BLK):
            src = pl.ds(hoff + j * _BLK, _BLK)
            dst = pl.ds(j * _BLK, _BLK)
            g1.append(pltpu.async_copy(
                tab_hbm.at[idx_all.at[_IPOS, src]], pos_rows.at[dst], sem_g))
            g1.append(pltpu.async_copy(
                tab_hbm.at[idx_all.at[_INEG, src]], neg_rows.at[dst], sem_g))
            g1.append(pltpu.async_copy(
                tab_hbm.at[idx_all.at[0, src]], acc.at[dst], sem_a))
        for c in g1:
            c.wait()
        g2 = []
        for j in range(_HALF // _BLK):
            src = pl.ds(hoff + j * _BLK, _BLK)
            dst = pl.ds(j * _BLK, _BLK)
            for w in range(1, _WIN):
                g2.append(pltpu.async_copy(
                    tab_hbm.at[idx_all.at[w, src]], acc.at[dst], sem_a,
                    add=True))
        for c in g2:
            c.wait()

        def elem(e, carry):
            pv = None
            nv = None
            for k in range(_D // 16):
                a = acc[e, pl.ds(_D + k * 16, 16)]
                p = pos_rows[e, pl.ds(k * 16, 16)] * a
                n = neg_rows[e, pl.ds(k * 16, 16)] * a
                pv = p if pv is None else pv + p
                nv = n if nv is None else nv + n
            row = (hoff + e) // 8
            lane = ((hoff + e) % 8) * 16
            stage_p[row, pl.ds(lane, 16)] = pv
            stage_n[row, pl.ds(lane, 16)] = nv
            return carry

        lax.fori_loop(0, _HALF, elem, 0)

    orow = wid * (_BPW // 8)
    o1 = pltpu.async_copy(stage_p, opos_hbm.at[pl.ds(orow, _BPW // 8)], sem_o)
    o2 = pltpu.async_copy(stage_n, oneg_hbm.at[pl.ds(orow, _BPW // 8)], sem_o)
    o1.wait()
    o2.wait()


_sc_cbow = functools.partial(
    pl.kernel,
    out_type=(jax.ShapeDtypeStruct((_B // 8, 128), jnp.float32),
              jax.ShapeDtypeStruct((_B // 8, 128), jnp.float32)),
    mesh=plsc.VectorSubcoreMesh(core_axis_name="c", subcore_axis_name="s",
                                num_cores=_NC, num_subcores=_NS),
    scratch_types=[
        pltpu.VMEM((_WIN + 2, _BPW), jnp.int32),
        pltpu.VMEM((_HALF, 2 * _D), jnp.float32),
        pltpu.VMEM((_HALF, 2 * _D), jnp.float32),
        pltpu.VMEM((_HALF, 2 * _D), jnp.float32),
        pltpu.VMEM((_BPW // 8, 128), jnp.float32),
        pltpu.VMEM((_BPW // 8, 128), jnp.float32),
        pltpu.SemaphoreType.DMA,
        pltpu.SemaphoreType.DMA,
        pltpu.SemaphoreType.DMA,
        pltpu.SemaphoreType.DMA,
    ],
    compiler_params=pltpu.CompilerParams(use_tc_tiling_on_sc=True),
)(_sc_body)


_N = 1000001
_CH = 4096
_NCHUNK = -(-_N // _CH)


def _tc_tab(twT_ref, cwT_ref, tab_ref):
    tab_ref[...] = jnp.concatenate(
        [twT_ref[...].T, cwT_ref[...].T], axis=1)


_build_tab = pl.pallas_call(
    _tc_tab,
    grid=(_NCHUNK,),
    in_specs=[pl.BlockSpec((_D, _CH), lambda i: (0, i)),
              pl.BlockSpec((_D, _CH), lambda i: (0, i))],
    out_specs=pl.BlockSpec((_CH, 2 * _D), lambda i: (i, 0)),
    out_shape=jax.ShapeDtypeStruct((_N, 2 * _D), jnp.float32),
)


def _tc_finish(pp_ref, np_ref, out_ref):
    l = lax.broadcasted_iota(jnp.int32, (128, 8), 0)
    s = lax.broadcasted_iota(jnp.int32, (128, 8), 1)
    m = (l // 16 == s).astype(jnp.float32)
    ps = jnp.dot(pp_ref[...], m) * (1.0 / _WIN) + _EPS
    ns = jnp.dot(np_ref[...], m) * (1.0 / _WIN) + _EPS
    pos_score = -jax.nn.log_sigmoid(ps)
    neg_score = -jax.nn.log_sigmoid(1.0 - ns)
    out_ref[0, 0] = (jnp.sum(pos_score) + jnp.sum(neg_score)) * (1.0 / _B)


def kernel(pos_nodes, neg_nodes, context_nodes, target_weight, context_weight):
    pos = pos_nodes.astype(jnp.int32)
    neg = neg_nodes.astype(jnp.int32)
    ctxT = context_nodes.astype(jnp.int32).T
    tab = _build_tab(target_weight.T, context_weight.T)

    pp, nn = _sc_cbow(pos, neg, ctxT, tab)

    loss = pl.pallas_call(
        _tc_finish,
        out_shape=jax.ShapeDtypeStruct((1, 1), jnp.float32),
        out_specs=pl.BlockSpec(memory_space=pltpu.SMEM),
    )(pp, nn)
    return loss[0, 0]


# speedup vs baseline: 2.9092x; 1.1910x over previous
import functools

import jax
import jax.numpy as jnp
from jax import lax
from jax.experimental import pallas as pl
from jax.experimental.pallas import tpu as pltpu
from jax.experimental.pallas import tpu_sc as plsc

_EPS = 1e-15
_B = 16384
_D = 64
_WIN = 20
_NC = 2
_NS = 16
_NW = _NC * _NS
_BPW = _B // _NW
_BLK = 128
_HALF = 256
_IPOS = _WIN
_INEG = _WIN + 1


def _sc_body(pos_hbm, neg_hbm, ctxT_hbm, tab_hbm, opos_hbm, oneg_hbm,
             idx_all, pos_rows, neg_rows, acc, stage_p, stage_n,
             sem_idx, sem_g, sem_a, sem_o):
    wid = lax.axis_index("s") * _NC + lax.axis_index("c")
    base = wid * _BPW

    idx_cps = [
        pltpu.async_copy(pos_hbm.at[pl.ds(base, _BPW)], idx_all.at[_IPOS],
                         sem_idx),
        pltpu.async_copy(neg_hbm.at[pl.ds(base, _BPW)], idx_all.at[_INEG],
                         sem_idx),
    ]
    for w in range(_WIN):
        idx_cps.append(pltpu.async_copy(
            ctxT_hbm.at[w, pl.ds(base, _BPW)], idx_all.at[w], sem_idx))
    for c in idx_cps:
        c.wait()

    for half in range(_BPW // _HALF):
        hoff = half * _HALF
        g1 = []
        for j in range(_HALF // _BLK):
            src = pl.ds(hoff + j * _BLK, _BLK)
            dst = pl.ds(j * _BLK, _BLK)
            g1.append(pltpu.async_copy(
                tab_hbm.at[idx_all.at[_IPOS, src]], pos_rows.at[dst], sem_g))
            g1.append(pltpu.async_copy(
                tab_hbm.at[idx_all.at[_INEG, src]], neg_rows.at[dst], sem_g))
            g1.append(pltpu.async_copy(
                tab_hbm.at[idx_all.at[0, src]], acc.at[dst], sem_a))
        for c in g1:
            c.wait()
        g2 = []
        for j in range(_HALF // _BLK):
            src = pl.ds(hoff + j * _BLK, _BLK)
            dst = pl.ds(j * _BLK, _BLK)
            for w in range(1, _WIN):
                g2.append(pltpu.async_copy(
                    tab_hbm.at[idx_all.at[w, src]], acc.at[dst], sem_a,
                    add=True))
        for c in g2:
            c.wait()

        def elem(e, carry):
            pv = None
            nv = None
            for k in range(_D // 16):
                a = acc[e, pl.ds(_D + k * 16, 16)]
                p = pos_rows[e, pl.ds(k * 16, 16)] * a
                n = neg_rows[e, pl.ds(k * 16, 16)] * a
                pv = p if pv is None else pv + p
                nv = n if nv is None else nv + n
            row = (hoff + e) // 8
            lane = ((hoff + e) % 8) * 16
            stage_p[row, pl.ds(lane, 16)] = pv
            stage_n[row, pl.ds(lane, 16)] = nv
            return carry

        lax.fori_loop(0, _HALF, elem, 0)

    orow = wid * (_BPW // 8)
    o1 = pltpu.async_copy(stage_p, opos_hbm.at[pl.ds(orow, _BPW // 8)], sem_o)
    o2 = pltpu.async_copy(stage_n, oneg_hbm.at[pl.ds(orow, _BPW // 8)], sem_o)
    o1.wait()
    o2.wait()


_sc_cbow = functools.partial(
    pl.kernel,
    out_type=(jax.ShapeDtypeStruct((_B // 8, 128), jnp.float32),
              jax.ShapeDtypeStruct((_B // 8, 128), jnp.float32)),
    mesh=plsc.VectorSubcoreMesh(core_axis_name="c", subcore_axis_name="s",
                                num_cores=_NC, num_subcores=_NS),
    scratch_types=[
        pltpu.VMEM((_WIN + 2, _BPW), jnp.int32),
        pltpu.VMEM((_HALF, 2 * _D), jnp.float32),
        pltpu.VMEM((_HALF, 2 * _D), jnp.float32),
        pltpu.VMEM((_HALF, 2 * _D), jnp.float32),
        pltpu.VMEM((_BPW // 8, 128), jnp.float32),
        pltpu.VMEM((_BPW // 8, 128), jnp.float32),
        pltpu.SemaphoreType.DMA,
        pltpu.SemaphoreType.DMA,
        pltpu.SemaphoreType.DMA,
        pltpu.SemaphoreType.DMA,
    ],
    compiler_params=pltpu.CompilerParams(use_tc_tiling_on_sc=True),
)(_sc_body)


_N = 1000001
_CH = 16384
_NCHUNK = -(-_N // _CH)


def _tc_tab(twT_ref, cwT_ref, tab_ref):
    tab_ref[...] = jnp.concatenate(
        [twT_ref[...].T, cwT_ref[...].T], axis=1)


_build_tab = pl.pallas_call(
    _tc_tab,
    grid=(_NCHUNK,),
    in_specs=[pl.BlockSpec((_D, _CH), lambda i: (0, i)),
              pl.BlockSpec((_D, _CH), lambda i: (0, i))],
    out_specs=pl.BlockSpec((_CH, 2 * _D), lambda i: (i, 0)),
    out_shape=jax.ShapeDtypeStruct((_N, 2 * _D), jnp.float32),
)


def _tc_finish(pp_ref, np_ref, out_ref):
    l = lax.broadcasted_iota(jnp.int32, (128, 8), 0)
    s = lax.broadcasted_iota(jnp.int32, (128, 8), 1)
    m = (l // 16 == s).astype(jnp.float32)
    ps = jnp.dot(pp_ref[...], m) * (1.0 / _WIN) + _EPS
    ns = jnp.dot(np_ref[...], m) * (1.0 / _WIN) + _EPS
    pos_score = -jax.nn.log_sigmoid(ps)
    neg_score = -jax.nn.log_sigmoid(1.0 - ns)
    out_ref[0, 0] = (jnp.sum(pos_score) + jnp.sum(neg_score)) * (1.0 / _B)


def kernel(pos_nodes, neg_nodes, context_nodes, target_weight, context_weight):
    pos = pos_nodes.astype(jnp.int32)
    neg = neg_nodes.astype(jnp.int32)
    ctxT = context_nodes.astype(jnp.int32).T
    tab = _build_tab(target_weight.T, context_weight.T)

    pp, nn = _sc_cbow(pos, neg, ctxT, tab)

    loss = pl.pallas_call(
        _tc_finish,
        out_shape=jax.ShapeDtypeStruct((1, 1), jnp.float32),
        out_specs=pl.BlockSpec(memory_space=pltpu.SMEM),
    )(pp, nn)
    return loss[0, 0]


# speedup vs baseline: 2.9155x; 1.0022x over previous
import functools

import jax
import jax.numpy as jnp
from jax import lax
from jax.experimental import pallas as pl
from jax.experimental.pallas import tpu as pltpu
from jax.experimental.pallas import tpu_sc as plsc

_EPS = 1e-15
_B = 16384
_D = 64
_WIN = 20
_NC = 2
_NS = 16
_NW = _NC * _NS
_BPW = _B // _NW
_BLK = 128
_HALF = 256
_IPOS = _WIN
_INEG = _WIN + 1


def _sc_body(pos_hbm, neg_hbm, ctxT_hbm, tab_hbm, opos_hbm, oneg_hbm,
             idx_all, pos_rows, neg_rows, acc, stage_p, stage_n,
             sem_idx, sem_g, sem_a, sem_o):
    wid = lax.axis_index("s") * _NC + lax.axis_index("c")
    base = wid * _BPW

    idx_cps = [
        pltpu.async_copy(pos_hbm.at[pl.ds(base, _BPW)], idx_all.at[_IPOS],
                         sem_idx),
        pltpu.async_copy(neg_hbm.at[pl.ds(base, _BPW)], idx_all.at[_INEG],
                         sem_idx),
    ]
    for w in range(_WIN):
        idx_cps.append(pltpu.async_copy(
            ctxT_hbm.at[w, pl.ds(base, _BPW)], idx_all.at[w], sem_idx))
    for c in idx_cps:
        c.wait()

    for half in range(_BPW // _HALF):
        hoff = half * _HALF
        g1 = []
        for j in range(_HALF // _BLK):
            src = pl.ds(hoff + j * _BLK, _BLK)
            dst = pl.ds(j * _BLK, _BLK)
            g1.append(pltpu.async_copy(
                tab_hbm.at[idx_all.at[_IPOS, src]], pos_rows.at[dst], sem_g))
            g1.append(pltpu.async_copy(
                tab_hbm.at[idx_all.at[_INEG, src]], neg_rows.at[dst], sem_g))
            g1.append(pltpu.async_copy(
                tab_hbm.at[idx_all.at[0, src]], acc.at[dst], sem_a))
        for c in g1:
            c.wait()
        g2 = []
        for j in range(_HALF // _BLK):
            src = pl.ds(hoff + j * _BLK, _BLK)
            dst = pl.ds(j * _BLK, _BLK)
            for w in range(1, _WIN):
                g2.append(pltpu.async_copy(
                    tab_hbm.at[idx_all.at[w, src]], acc.at[dst], sem_a,
                    add=True))
        for c in g2:
            c.wait()

        def elem(e, carry):
            pv = None
            nv = None
            for k in range(_D // 16):
                a = acc[e, pl.ds(_D + k * 16, 16)]
                p = pos_rows[e, pl.ds(k * 16, 16)] * a
                n = neg_rows[e, pl.ds(k * 16, 16)] * a
                pv = p if pv is None else pv + p
                nv = n if nv is None else nv + n
            row = (hoff + e) // 8
            lane = ((hoff + e) % 8) * 16
            stage_p[row, pl.ds(lane, 16)] = pv
            stage_n[row, pl.ds(lane, 16)] = nv
            return carry

        lax.fori_loop(0, _HALF, elem, 0)

    orow = wid * (_BPW // 8)
    o1 = pltpu.async_copy(stage_p, opos_hbm.at[pl.ds(orow, _BPW // 8)], sem_o)
    o2 = pltpu.async_copy(stage_n, oneg_hbm.at[pl.ds(orow, _BPW // 8)], sem_o)
    o1.wait()
    o2.wait()


_sc_cbow = functools.partial(
    pl.kernel,
    out_type=(jax.ShapeDtypeStruct((_B // 8, 128), jnp.float32),
              jax.ShapeDtypeStruct((_B // 8, 128), jnp.float32)),
    mesh=plsc.VectorSubcoreMesh(core_axis_name="c", subcore_axis_name="s",
                                num_cores=_NC, num_subcores=_NS),
    scratch_types=[
        pltpu.VMEM((_WIN + 2, _BPW), jnp.int32),
        pltpu.VMEM((_HALF, 2 * _D), jnp.float32),
        pltpu.VMEM((_HALF, 2 * _D), jnp.float32),
        pltpu.VMEM((_HALF, 2 * _D), jnp.float32),
        pltpu.VMEM((_BPW // 8, 128), jnp.float32),
        pltpu.VMEM((_BPW // 8, 128), jnp.float32),
        pltpu.SemaphoreType.DMA,
        pltpu.SemaphoreType.DMA,
        pltpu.SemaphoreType.DMA,
        pltpu.SemaphoreType.DMA,
    ],
    compiler_params=pltpu.CompilerParams(use_tc_tiling_on_sc=True),
)(_sc_body)


_N = 1000001
_CH = 16384
_NCHUNK = -(-_N // _CH)


def _tc_tab(twT_ref, cwT_ref, tab_ref):
    for k in range(_CH // 4096):
        sl = pl.ds(k * 4096, 4096)
        tab_ref[sl, 0:_D] = twT_ref[:, sl].T
        tab_ref[sl, _D:2 * _D] = cwT_ref[:, sl].T


_build_tab = pl.pallas_call(
    _tc_tab,
    grid=(_NCHUNK,),
    in_specs=[pl.BlockSpec((_D, _CH), lambda i: (0, i)),
              pl.BlockSpec((_D, _CH), lambda i: (0, i))],
    out_specs=pl.BlockSpec((_CH, 2 * _D), lambda i: (i, 0)),
    out_shape=jax.ShapeDtypeStruct((_N, 2 * _D), jnp.float32),
    compiler_params=pltpu.CompilerParams(vmem_limit_bytes=110 * 1024 * 1024),
)


def _tc_finish(pp_ref, np_ref, out_ref):
    l = lax.broadcasted_iota(jnp.int32, (128, 8), 0)
    s = lax.broadcasted_iota(jnp.int32, (128, 8), 1)
    m = (l // 16 == s).astype(jnp.float32)
    ps = jnp.dot(pp_ref[...], m) * (1.0 / _WIN) + _EPS
    ns = jnp.dot(np_ref[...], m) * (1.0 / _WIN) + _EPS
    pos_score = -jax.nn.log_sigmoid(ps)
    neg_score = -jax.nn.log_sigmoid(1.0 - ns)
    out_ref[0, 0] = (jnp.sum(pos_score) + jnp.sum(neg_score)) * (1.0 / _B)


def kernel(pos_nodes, neg_nodes, context_nodes, target_weight, context_weight):
    pos = pos_nodes.astype(jnp.int32)
    neg = neg_nodes.astype(jnp.int32)
    ctxT = context_nodes.astype(jnp.int32).T
    tab = _build_tab(target_weight.T, context_weight.T)

    pp, nn = _sc_cbow(pos, neg, ctxT, tab)

    loss = pl.pallas_call(
        _tc_finish,
        out_shape=jax.ShapeDtypeStruct((1, 1), jnp.float32),
        out_specs=pl.BlockSpec(memory_space=pltpu.SMEM),
    )(pp, nn)
    return loss[0, 0]


# speedup vs baseline: 2.9272x; 1.0040x over previous
import functools

import jax
import jax.numpy as jnp
from jax import lax
from jax.experimental import pallas as pl
from jax.experimental.pallas import tpu as pltpu
from jax.experimental.pallas import tpu_sc as plsc

_EPS = 1e-15
_B = 16384
_D = 64
_WIN = 20
_NC = 2
_NS = 16
_NW = _NC * _NS
_BPW = _B // _NW
_BLK = 128
_HALF = 256
_IPOS = _WIN
_INEG = _WIN + 1


def _sc_body(pos_hbm, neg_hbm, ctxT_hbm, tab_hbm, opos_hbm, oneg_hbm,
             idx_all, pos_rows, neg_rows, acc, stage_p, stage_n,
             sem_idx, sem_g, sem_a, sem_o):
    wid = lax.axis_index("s") * _NC + lax.axis_index("c")
    base = wid * _BPW

    idx_cps = [
        pltpu.async_copy(pos_hbm.at[pl.ds(base, _BPW)], idx_all.at[_IPOS],
                         sem_idx),
        pltpu.async_copy(neg_hbm.at[pl.ds(base, _BPW)], idx_all.at[_INEG],
                         sem_idx),
    ]
    for w in range(_WIN):
        idx_cps.append(pltpu.async_copy(
            ctxT_hbm.at[w, pl.ds(base, _BPW)], idx_all.at[w], sem_idx))
    for c in idx_cps:
        c.wait()

    for half in range(_BPW // _HALF):
        hoff = half * _HALF
        g1 = []
        for j in range(_HALF // _BLK):
            src = pl.ds(hoff + j * _BLK, _BLK)
            dst = pl.ds(j * _BLK, _BLK)
            g1.append(pltpu.async_copy(
                tab_hbm.at[idx_all.at[_IPOS, src]], pos_rows.at[dst], sem_g))
            g1.append(pltpu.async_copy(
                tab_hbm.at[idx_all.at[_INEG, src]], neg_rows.at[dst], sem_g))
            g1.append(pltpu.async_copy(
                tab_hbm.at[idx_all.at[0, src]], acc.at[dst], sem_a))
        for c in g1:
            c.wait()
        g2 = []
        for j in range(_HALF // _BLK):
            src = pl.ds(hoff + j * _BLK, _BLK)
            dst = pl.ds(j * _BLK, _BLK)
            for w in range(1, _WIN):
                g2.append(pltpu.async_copy(
                    tab_hbm.at[idx_all.at[w, src]], acc.at[dst], sem_a,
                    add=True))
        for c in g2:
            c.wait()

        def elem(e, carry):
            pv = None
            nv = None
            for k in range(_D // 16):
                a = acc[e, pl.ds(_D + k * 16, 16)]
                p = pos_rows[e, pl.ds(k * 16, 16)] * a
                n = neg_rows[e, pl.ds(k * 16, 16)] * a
                pv = p if pv is None else pv + p
                nv = n if nv is None else nv + n
            row = (hoff + e) // 8
            lane = ((hoff + e) % 8) * 16
            stage_p[row, pl.ds(lane, 16)] = pv
            stage_n[row, pl.ds(lane, 16)] = nv
            return carry

        lax.fori_loop(0, _HALF, elem, 0)

    orow = wid * (_BPW // 8)
    o1 = pltpu.async_copy(stage_p, opos_hbm.at[pl.ds(orow, _BPW // 8)], sem_o)
    o2 = pltpu.async_copy(stage_n, oneg_hbm.at[pl.ds(orow, _BPW // 8)], sem_o)
    o1.wait()
    o2.wait()


_sc_cbow = functools.partial(
    pl.kernel,
    out_type=(jax.ShapeDtypeStruct((_B // 8, 128), jnp.float32),
              jax.ShapeDtypeStruct((_B // 8, 128), jnp.float32)),
    mesh=plsc.VectorSubcoreMesh(core_axis_name="c", subcore_axis_name="s",
                                num_cores=_NC, num_subcores=_NS),
    scratch_types=[
        pltpu.VMEM((_WIN + 2, _BPW), jnp.int32),
        pltpu.VMEM((_HALF, 2 * _D), jnp.float32),
        pltpu.VMEM((_HALF, 2 * _D), jnp.float32),
        pltpu.VMEM((_HALF, 2 * _D), jnp.float32),
        pltpu.VMEM((_BPW // 8, 128), jnp.float32),
        pltpu.VMEM((_BPW // 8, 128), jnp.float32),
        pltpu.SemaphoreType.DMA,
        pltpu.SemaphoreType.DMA,
        pltpu.SemaphoreType.DMA,
        pltpu.SemaphoreType.DMA,
    ],
    compiler_params=pltpu.CompilerParams(use_tc_tiling_on_sc=True),
)(_sc_body)


_N = 1000001
_CH = 24576
_NCHUNK = -(-_N // _CH)


def _tc_tab(twT_ref, cwT_ref, tab_ref):
    for k in range(_CH // 4096):
        sl = pl.ds(k * 4096, 4096)
        tab_ref[sl, 0:_D] = twT_ref[:, sl].T
        tab_ref[sl, _D:2 * _D] = cwT_ref[:, sl].T


_build_tab = pl.pallas_call(
    _tc_tab,
    grid=(_NCHUNK,),
    in_specs=[pl.BlockSpec((_D, _CH), lambda i: (0, i)),
              pl.BlockSpec((_D, _CH), lambda i: (0, i))],
    out_specs=pl.BlockSpec((_CH, 2 * _D), lambda i: (i, 0)),
    out_shape=jax.ShapeDtypeStruct((_N, 2 * _D), jnp.float32),
    compiler_params=pltpu.CompilerParams(vmem_limit_bytes=110 * 1024 * 1024),
)


def _tc_finish(pp_ref, np_ref, out_ref):
    l = lax.broadcasted_iota(jnp.int32, (128, 8), 0)
    s = lax.broadcasted_iota(jnp.int32, (128, 8), 1)
    m = (l // 16 == s).astype(jnp.float32)
    ps = jnp.dot(pp_ref[...], m) * (1.0 / _WIN) + _EPS
    ns = jnp.dot(np_ref[...], m) * (1.0 / _WIN) + _EPS
    pos_score = -jax.nn.log_sigmoid(ps)
    neg_score = -jax.nn.log_sigmoid(1.0 - ns)
    out_ref[0, 0] = (jnp.sum(pos_score) + jnp.sum(neg_score)) * (1.0 / _B)


def kernel(pos_nodes, neg_nodes, context_nodes, target_weight, context_weight):
    pos = pos_nodes.astype(jnp.int32)
    neg = neg_nodes.astype(jnp.int32)
    ctxT = context_nodes.astype(jnp.int32).T
    tab = _build_tab(target_weight.T, context_weight.T)

    pp, nn = _sc_cbow(pos, neg, ctxT, tab)

    loss = pl.pallas_call(
        _tc_finish,
        out_shape=jax.ShapeDtypeStruct((1, 1), jnp.float32),
        out_specs=pl.BlockSpec(memory_space=pltpu.SMEM),
    )(pp, nn)
    return loss[0, 0]


# speedup vs baseline: 2.9296x; 1.0008x over previous
import functools

import jax
import jax.numpy as jnp
from jax import lax
from jax.experimental import pallas as pl
from jax.experimental.pallas import tpu as pltpu
from jax.experimental.pallas import tpu_sc as plsc

_EPS = 1e-15
_B = 16384
_D = 64
_WIN = 20
_NC = 2
_NS = 16
_NW = _NC * _NS
_BPW = _B // _NW
_BLK = 128
_HALF = 256
_IPOS = _WIN
_INEG = _WIN + 1


def _sc_body(pos_hbm, neg_hbm, ctxT_hbm, tab_hbm, opos_hbm, oneg_hbm,
             idx_all, pos_rows, neg_rows, acc, stage_p, stage_n,
             sem_idx, sem_g, sem_a, sem_o):
    wid = lax.axis_index("s") * _NC + lax.axis_index("c")
    base = wid * _BPW

    idx_cps = [
        pltpu.async_copy(pos_hbm.at[pl.ds(base, _BPW)], idx_all.at[_IPOS],
                         sem_idx),
        pltpu.async_copy(neg_hbm.at[pl.ds(base, _BPW)], idx_all.at[_INEG],
                         sem_idx),
    ]
    for w in range(_WIN):
        idx_cps.append(pltpu.async_copy(
            ctxT_hbm.at[w, pl.ds(base, _BPW)], idx_all.at[w], sem_idx))
    for c in idx_cps:
        c.wait()

    nblk = _BPW // _BLK

    def fire_g1(b):
        src = pl.ds(b * _BLK, _BLK)
        dst = pl.ds((b % 2) * _BLK, _BLK)
        return [
            pltpu.async_copy(
                tab_hbm.at[idx_all.at[_IPOS, src]], pos_rows.at[dst], sem_g),
            pltpu.async_copy(
                tab_hbm.at[idx_all.at[_INEG, src]], neg_rows.at[dst], sem_g),
            pltpu.async_copy(
                tab_hbm.at[idx_all.at[0, src]], acc.at[dst], sem_a),
        ]

    def fire_g2(b):
        src = pl.ds(b * _BLK, _BLK)
        dst = pl.ds((b % 2) * _BLK, _BLK)
        return [
            pltpu.async_copy(
                tab_hbm.at[idx_all.at[w, src]], acc.at[dst], sem_a, add=True)
            for w in range(1, _WIN)
        ]

    def compute(b):
        boff = (b % 2) * _BLK

        def elem(e, carry):
            pv = None
            nv = None
            for k in range(_D // 16):
                a = acc[boff + e, pl.ds(_D + k * 16, 16)]
                p = pos_rows[boff + e, pl.ds(k * 16, 16)] * a
                n = neg_rows[boff + e, pl.ds(k * 16, 16)] * a
                pv = p if pv is None else pv + p
                nv = n if nv is None else nv + n
            row = (b * _BLK + e) // 8
            lane = ((b * _BLK + e) % 8) * 16
            stage_p[row, pl.ds(lane, 16)] = pv
            stage_n[row, pl.ds(lane, 16)] = nv
            return carry

        lax.fori_loop(0, _BLK, elem, 0)

    pend_g1 = {0: fire_g1(0)}
    pend_g2 = {}
    for b in range(nblk):
        for c in pend_g1.pop(b):
            c.wait()
        pend_g2[b] = fire_g2(b)
        if b + 1 < nblk:
            pend_g1[b + 1] = fire_g1(b + 1)
        for c in pend_g2.pop(b):
            c.wait()
        compute(b)

    orow = wid * (_BPW // 8)
    o1 = pltpu.async_copy(stage_p, opos_hbm.at[pl.ds(orow, _BPW // 8)], sem_o)
    o2 = pltpu.async_copy(stage_n, oneg_hbm.at[pl.ds(orow, _BPW // 8)], sem_o)
    o1.wait()
    o2.wait()


_sc_cbow = functools.partial(
    pl.kernel,
    out_type=(jax.ShapeDtypeStruct((_B // 8, 128), jnp.float32),
              jax.ShapeDtypeStruct((_B // 8, 128), jnp.float32)),
    mesh=plsc.VectorSubcoreMesh(core_axis_name="c", subcore_axis_name="s",
                                num_cores=_NC, num_subcores=_NS),
    scratch_types=[
        pltpu.VMEM((_WIN + 2, _BPW), jnp.int32),
        pltpu.VMEM((_HALF, 2 * _D), jnp.float32),
        pltpu.VMEM((_HALF, 2 * _D), jnp.float32),
        pltpu.VMEM((_HALF, 2 * _D), jnp.float32),
        pltpu.VMEM((_BPW // 8, 128), jnp.float32),
        pltpu.VMEM((_BPW // 8, 128), jnp.float32),
        pltpu.SemaphoreType.DMA,
        pltpu.SemaphoreType.DMA,
        pltpu.SemaphoreType.DMA,
        pltpu.SemaphoreType.DMA,
    ],
    compiler_params=pltpu.CompilerParams(use_tc_tiling_on_sc=True),
)(_sc_body)


_N = 1000001
_CH = 24576
_NCHUNK = -(-_N // _CH)


def _tc_tab(twT_ref, cwT_ref, tab_ref):
    for k in range(_CH // 4096):
        sl = pl.ds(k * 4096, 4096)
        tab_ref[sl, 0:_D] = twT_ref[:, sl].T
        tab_ref[sl, _D:2 * _D] = cwT_ref[:, sl].T


_build_tab = pl.pallas_call(
    _tc_tab,
    grid=(_NCHUNK,),
    in_specs=[pl.BlockSpec((_D, _CH), lambda i: (0, i)),
              pl.BlockSpec((_D, _CH), lambda i: (0, i))],
    out_specs=pl.BlockSpec((_CH, 2 * _D), lambda i: (i, 0)),
    out_shape=jax.ShapeDtypeStruct((_N, 2 * _D), jnp.float32),
    compiler_params=pltpu.CompilerParams(vmem_limit_bytes=110 * 1024 * 1024),
)


def _tc_finish(pp_ref, np_ref, out_ref):
    l = lax.broadcasted_iota(jnp.int32, (128, 8), 0)
    s = lax.broadcasted_iota(jnp.int32, (128, 8), 1)
    m = (l // 16 == s).astype(jnp.float32)
    ps = jnp.dot(pp_ref[...], m) * (1.0 / _WIN) + _EPS
    ns = jnp.dot(np_ref[...], m) * (1.0 / _WIN) + _EPS
    pos_score = -jax.nn.log_sigmoid(ps)
    neg_score = -jax.nn.log_sigmoid(1.0 - ns)
    out_ref[0, 0] = (jnp.sum(pos_score) + jnp.sum(neg_score)) * (1.0 / _B)


def kernel(pos_nodes, neg_nodes, context_nodes, target_weight, context_weight):
    pos = pos_nodes.astype(jnp.int32)
    neg = neg_nodes.astype(jnp.int32)
    ctxT = context_nodes.astype(jnp.int32).T
    tab = _build_tab(target_weight.T, context_weight.T)

    pp, nn = _sc_cbow(pos, neg, ctxT, tab)

    loss = pl.pallas_call(
        _tc_finish,
        out_shape=jax.ShapeDtypeStruct((1, 1), jnp.float32),
        out_specs=pl.BlockSpec(memory_space=pltpu.SMEM),
    )(pp, nn)
    return loss[0, 0]
